# baseline (speedup 1.0000x reference)
import functools

import jax
import jax.numpy as jnp
from jax import lax
from jax.experimental import pallas as pl
from jax.experimental.pallas import tpu as pltpu
from jax.experimental.pallas import tpu_sc as plsc

B = 16
N = 65536
FEAT = 64
INV_T = 1.0 / 0.07

NUM_WORKERS = 32
ROWS = 6 * B
ROUNDS = ROWS // NUM_WORKERS
CHUNK = 8192
N_CHUNKS = N // CHUNK


def _qt_kernel(fz_ref, fm_ref, fr_ref, bz_ref, bm_ref, br_ref, out_ref):
    fz = fz_ref[...] * INV_T
    fm = fm_ref[...] * INV_T
    fr = fr_ref[...] * INV_T
    dims = (((1,), (0,)), ((), ()))
    blk = bz_ref.shape[1]
    qz = lax.dot_general(
        fz, bz_ref[...], dims, preferred_element_type=jnp.float32)
    qm = lax.dot_general(
        fm, bm_ref[...], dims, preferred_element_type=jnp.float32)
    qr = lax.dot_general(
        fr, br_ref[...], dims, preferred_element_type=jnp.float32)
    out_ref[0:32] = qz.reshape(32, blk // 128, 128)
    out_ref[32:64] = qm.reshape(32, blk // 128, 128)
    out_ref[64:96] = qr.reshape(32, blk // 128, 128)


def _compute_qt(f_z, f_m, f_r, bank_zt, bank_mt, bank_rt):
    blk = 4096
    grid = (N // blk,)
    return pl.pallas_call(
        _qt_kernel,
        grid=grid,
        in_specs=[
            pl.BlockSpec((32, FEAT), lambda i: (0, 0)),
            pl.BlockSpec((32, FEAT), lambda i: (0, 0)),
            pl.BlockSpec((32, FEAT), lambda i: (0, 0)),
            pl.BlockSpec((FEAT, blk), lambda i: (0, i)),
            pl.BlockSpec((FEAT, blk), lambda i: (0, i)),
            pl.BlockSpec((FEAT, blk), lambda i: (0, i)),
        ],
        out_specs=pl.BlockSpec((ROWS, blk // 128, 128), lambda i: (0, i, 0)),
        out_shape=jax.ShapeDtypeStruct((ROWS, N // 128, 128), jnp.float32),
    )(f_z, f_m, f_r, bank_zt, bank_mt, bank_rt)


def _sc_body(qt_hbm, idx_hbm, o0, o1, o2, o3, o4, o5,
             q_v, idx_v, o_v, sem_q, sem_o0, sem_o1):
    c = lax.axis_index("c")
    s = lax.axis_index("s")
    wid = s * 2 + c
    b = lax.rem(wid, B)
    hi = wid >= B
    outs_lo = (o0, o1, o2)
    outs_hi = (o5, o3, o4)
    out_sems = (sem_o0, sem_o1)
    iot = lax.iota(jnp.int32, 16)

    def out_drain(t, ch, bf):
        col = pl.ds(ch * (CHUNK // 128), CHUNK // 128)

        @pl.when(jnp.logical_not(hi))
        def _():
            pltpu.make_async_copy(o_v.at[bf], outs_lo[t].at[b, col],
                                  out_sems[bf]).wait()

        @pl.when(hi)
        def _():
            pltpu.make_async_copy(o_v.at[bf], outs_hi[t].at[b, col],
                                  out_sems[bf]).wait()

    pltpu.sync_copy(idx_hbm.at[b], idx_v)

    out_pending = [None, None]
    for t in range(ROUNDS):
        r = t * NUM_WORKERS + wid
        pltpu.sync_copy(qt_hbm.at[r], q_v)
        for ch in range(N_CHUNKS):
            bf = ch % 2
            if out_pending[bf] is not None:
                out_drain(*out_pending[bf], bf)

            @plsc.parallel_loop(ch * (CHUNK // 2), (ch + 1) * (CHUNK // 2),
                                16, unroll=4)
            def _gather(p):
                w = idx_v[lax.shift_right_logical(p, 7),
                          pl.ds(lax.bitwise_and(p, 127), 16)]
                ia = lax.bitwise_and(w, 0xFFFF)
                ib = lax.shift_right_logical(w, 16)
                va = plsc.load_gather(
                    q_v, [lax.shift_right_logical(ia, 7),
                          lax.bitwise_and(ia, 127)])
                vb = plsc.load_gather(
                    q_v, [lax.shift_right_logical(ib, 7),
                          lax.bitwise_and(ib, 127)])
                pa = 2 * (p - ch * (CHUNK // 2)) + 2 * iot
                pb = pa + 1
                plsc.store_scatter(
                    o_v.at[bf], [lax.shift_right_logical(pa, 7),
                                 lax.bitwise_and(pa, 127)], va)
                plsc.store_scatter(
                    o_v.at[bf], [lax.shift_right_logical(pb, 7),
                                 lax.bitwise_and(pb, 127)], vb)

            col = pl.ds(ch * (CHUNK // 128), CHUNK // 128)

            @pl.when(jnp.logical_not(hi))
            def _():
                pltpu.async_copy(o_v.at[bf], outs_lo[t].at[b, col],
                                 out_sems[bf])

            @pl.when(hi)
            def _():
                pltpu.async_copy(o_v.at[bf], outs_hi[t].at[b, col],
                                 out_sems[bf])

            out_pending[bf] = (t, ch)
    for bf in range(2):
        if out_pending[bf] is not None:
            out_drain(*out_pending[bf], bf)


def _sc_gather(qt, idx16):
    mesh = plsc.VectorSubcoreMesh(core_axis_name="c", subcore_axis_name="s")
    out_t = jax.ShapeDtypeStruct((B, N // 128, 128), jnp.float32)
    fn = functools.partial(
        pl.kernel,
        mesh=mesh,
        out_type=(out_t,) * 6,
        scratch_types=[
            pltpu.VMEM((N // 128, 128), jnp.float32),
            pltpu.VMEM((N // 256, 128), jnp.int32),
            pltpu.VMEM((2, CHUNK // 128, 128), jnp.float32),
            pltpu.SemaphoreType.DMA,
            pltpu.SemaphoreType.DMA,
            pltpu.SemaphoreType.DMA,
        ],
        compiler_params=pltpu.CompilerParams(needs_layout_passes=False),
    )(_sc_body)
    return fn(qt, idx16)


def kernel(fea_f, fea_fenzi, fea_fenmu, y, idx, memory_fringe, memory_fenzi,
           memory_fenmu):
    del y
    idx16 = idx.astype(jnp.uint16).reshape(B, N // 2, 2)
    idxp = lax.bitcast_convert_type(idx16, jnp.int32)
    idxp = idxp.reshape(B, N // 256, 128)
    f_z = jnp.concatenate([fea_f, fea_fenmu], axis=0)
    f_m = jnp.concatenate([fea_f, fea_fenzi], axis=0)
    f_r = jnp.concatenate([fea_fenzi, fea_fenmu], axis=0)

    qt = _compute_qt(f_z, f_m, f_r, memory_fenzi.T, memory_fenmu.T,
                     memory_fringe.T)
    outs = _sc_gather(qt, idxp)
    return tuple(o.reshape(B, N, 1) for o in outs)


# speedup vs baseline: 3.1337x; 3.1337x over previous
import functools

import jax
import jax.numpy as jnp
from jax import lax
from jax.experimental import pallas as pl
from jax.experimental.pallas import tpu as pltpu
from jax.experimental.pallas import tpu_sc as plsc

B = 16
N = 65536
FEAT = 64
INV_T = 1.0 / 0.07

NUM_WORKERS = 32
ROWS = 6 * B
ROUNDS = ROWS // NUM_WORKERS
CHUNK = 8192
N_CHUNKS = N // CHUNK


def _qt_kernel(fz_ref, fm_ref, fr_ref, bz_ref, bm_ref, br_ref, out_ref):
    fz = fz_ref[...] * INV_T
    fm = fm_ref[...] * INV_T
    fr = fr_ref[...] * INV_T
    dims = (((1,), (0,)), ((), ()))
    blk = bz_ref.shape[1]
    qz = lax.dot_general(
        fz, bz_ref[...], dims, preferred_element_type=jnp.float32)
    qm = lax.dot_general(
        fm, bm_ref[...], dims, preferred_element_type=jnp.float32)
    qr = lax.dot_general(
        fr, br_ref[...], dims, preferred_element_type=jnp.float32)
    out_ref[0:32] = qz.reshape(32, blk // 128, 128)
    out_ref[32:64] = qm.reshape(32, blk // 128, 128)
    out_ref[64:96] = qr.reshape(32, blk // 128, 128)


def _compute_qt(f_z, f_m, f_r, bank_zt, bank_mt, bank_rt):
    blk = 4096
    grid = (N // blk,)
    return pl.pallas_call(
        _qt_kernel,
        grid=grid,
        in_specs=[
            pl.BlockSpec((32, FEAT), lambda i: (0, 0)),
            pl.BlockSpec((32, FEAT), lambda i: (0, 0)),
            pl.BlockSpec((32, FEAT), lambda i: (0, 0)),
            pl.BlockSpec((FEAT, blk), lambda i: (0, i)),
            pl.BlockSpec((FEAT, blk), lambda i: (0, i)),
            pl.BlockSpec((FEAT, blk), lambda i: (0, i)),
        ],
        out_specs=pl.BlockSpec((ROWS, blk // 128, 128), lambda i: (0, i, 0)),
        out_shape=jax.ShapeDtypeStruct((ROWS, N // 128, 128), jnp.float32),
    )(f_z, f_m, f_r, bank_zt, bank_mt, bank_rt)


HROWS = N // 256
CROWS = CHUNK // 256


def _sc_body(qt_hbm, idx_hbm, o0, o1, o2, o3, o4, o5,
             q_v, idx_v, oa_v, ob_v, sem_a0, sem_a1, sem_b0, sem_b1):
    c = lax.axis_index("c")
    s = lax.axis_index("s")
    wid = s * 2 + c
    b = lax.rem(wid, B)
    hi = wid >= B
    outs_lo = (o0, o1, o2)
    outs_hi = (o5, o3, o4)
    a_sems = (sem_a0, sem_a1)
    b_sems = (sem_b0, sem_b1)

    def out_dma(t, ch, bf, make_only):
        col_a = pl.ds(ch * CROWS, CROWS)
        col_b = pl.ds(N // 256 + ch * CROWS, CROWS)

        @pl.when(jnp.logical_not(hi))
        def _():
            da = pltpu.make_async_copy(oa_v.at[bf], outs_lo[t].at[b, col_a],
                                       a_sems[bf])
            db = pltpu.make_async_copy(ob_v.at[bf], outs_lo[t].at[b, col_b],
                                       b_sems[bf])
            if make_only:
                da.wait()
                db.wait()
            else:
                da.start()
                db.start()

        @pl.when(hi)
        def _():
            da = pltpu.make_async_copy(oa_v.at[bf], outs_hi[t].at[b, col_a],
                                       a_sems[bf])
            db = pltpu.make_async_copy(ob_v.at[bf], outs_hi[t].at[b, col_b],
                                       b_sems[bf])
            if make_only:
                da.wait()
                db.wait()
            else:
                da.start()
                db.start()

    pltpu.sync_copy(idx_hbm.at[b], idx_v)

    out_pending = [None, None]
    for t in range(ROUNDS):
        r = t * NUM_WORKERS + wid
        pltpu.sync_copy(qt_hbm.at[r], q_v)
        for ch in range(N_CHUNKS):
            bf = ch % 2
            if out_pending[bf] is not None:
                out_dma(*out_pending[bf], bf, True)

            @plsc.parallel_loop(ch * CROWS, (ch + 1) * CROWS, 1, unroll=2)
            def _gather(p):
                lp = p - ch * CROWS
                for jj in range(0, 128, 16):
                    w = idx_v[p, pl.ds(jj, 16)]
                    ia = lax.bitwise_and(w, 0xFFFF)
                    ib = lax.shift_right_logical(w, 16)
                    oa_v[bf, lp, pl.ds(jj, 16)] = plsc.load_gather(
                        q_v, [lax.shift_right_logical(ia, 7),
                              lax.bitwise_and(ia, 127)])
                    ob_v[bf, lp, pl.ds(jj, 16)] = plsc.load_gather(
                        q_v, [lax.shift_right_logical(ib, 7),
                              lax.bitwise_and(ib, 127)])

            out_dma(t, ch, bf, False)
            out_pending[bf] = (t, ch)
    for bf in range(2):
        if out_pending[bf] is not None:
            out_dma(*out_pending[bf], bf, True)


def _sc_gather(qt, idxp):
    mesh = plsc.VectorSubcoreMesh(core_axis_name="c", subcore_axis_name="s")
    out_t = jax.ShapeDtypeStruct((B, N // 128, 128), jnp.float32)
    fn = functools.partial(
        pl.kernel,
        mesh=mesh,
        out_type=(out_t,) * 6,
        scratch_types=[
            pltpu.VMEM((N // 128, 128), jnp.float32),
            pltpu.VMEM((HROWS, 128), jnp.int32),
            pltpu.VMEM((2, CROWS, 128), jnp.float32),
            pltpu.VMEM((2, CROWS, 128), jnp.float32),
            pltpu.SemaphoreType.DMA,
            pltpu.SemaphoreType.DMA,
            pltpu.SemaphoreType.DMA,
            pltpu.SemaphoreType.DMA,
        ],
        compiler_params=pltpu.CompilerParams(needs_layout_passes=False),
    )(_sc_body)
    return fn(qt, idxp)


def kernel(fea_f, fea_fenzi, fea_fenmu, y, idx, memory_fringe, memory_fenzi,
           memory_fenmu):
    del y
    idx16 = idx.astype(jnp.uint16)
    idxp = lax.bitcast_convert_type(
        jnp.stack([idx16[:, :N // 2], idx16[:, N // 2:]], axis=-1),
        jnp.int32)
    idxp = idxp.reshape(B, N // 256, 128)
    f_z = jnp.concatenate([fea_f, fea_fenmu], axis=0)
    f_m = jnp.concatenate([fea_f, fea_fenzi], axis=0)
    f_r = jnp.concatenate([fea_fenzi, fea_fenmu], axis=0)

    qt = _compute_qt(f_z, f_m, f_r, memory_fenzi.T, memory_fenmu.T,
                     memory_fringe.T)
    outs = _sc_gather(qt, idxp)
    return tuple(o.reshape(B, N, 1) for o in outs)
